# baseline (speedup 1.0000x reference)
import functools

import jax
import jax.numpy as jnp
from jax import lax
from jax.experimental import pallas as pl
from jax.experimental.pallas import tpu as pltpu
from jax.experimental.pallas import tpu_sc as plsc

NC, NS, L = 2, 16, 16
NW = NC * NS
ROWS, COLS, D = 16384, 1024, 256
K = 8
RB = 8
NCHUNK = COLS // L
RPW = ROWS // NW
NBLK = RPW // RB
GIDX = RB * K
IDXPAD = GIDX + K

_mesh = plsc.VectorSubcoreMesh(core_axis_name="c", subcore_axis_name="s")


@functools.partial(
    pl.kernel,
    out_type=jax.ShapeDtypeStruct((ROWS, D), jnp.float32),
    mesh=_mesh,
    scratch_types=[
        pltpu.VMEM((RB, COLS), jnp.float32),
        pltpu.VMEM((RB, COLS), jnp.float32),
        pltpu.VMEM((IDXPAD,), jnp.int32),
        pltpu.VMEM((IDXPAD,), jnp.int32),
        pltpu.VMEM((IDXPAD, D), jnp.float32),
        pltpu.VMEM((IDXPAD, D), jnp.float32),
        pltpu.VMEM((RB, D), jnp.float32),
        pltpu.VMEM((RB, D), jnp.float32),
        pltpu.SemaphoreType.DMA,
        pltpu.SemaphoreType.DMA,
        pltpu.SemaphoreType.DMA,
        pltpu.SemaphoreType.DMA,
        pltpu.SemaphoreType.DMA,
        pltpu.SemaphoreType.DMA,
    ],
    compiler_params=pltpu.CompilerParams(needs_layout_passes=False),
)
def _nkq_sc(x_hbm, wt_hbm, out_hbm, xv0, xv1, ix0, ix1, rv0, rv1, ov0, ov1,
            xs0, xs1, gs0, gs1, os0, os1):
    x_v = (xv0, xv1)
    idx_v = (ix0, ix1)
    rows_v = (rv0, rv1)
    out_v = (ov0, ov1)
    xsem = (xs0, xs1)
    gsem = (gs0, gs1)
    osem = (os0, os1)

    wid = lax.axis_index("s") * NC + lax.axis_index("c")
    base0 = wid * RPW
    lanes = lax.iota(jnp.int32, L)
    store_mask = lanes < K
    neg_inf = jnp.full((L,), -jnp.inf, dtype=jnp.float32)
    zeros_i = jnp.zeros((L,), dtype=jnp.int32)

    for p in range(2):
        idx_v[p][pl.ds(IDXPAD - L, L)] = zeros_i

    def start_x(b, p):
        pltpu.async_copy(
            x_hbm.at[pl.ds(base0 + b * RB, RB)], x_v[p], xsem[p])

    def wait_x(b, p):
        pltpu.make_async_copy(
            x_hbm.at[pl.ds(base0 + b * RB, RB)], x_v[p], xsem[p]).wait()

    def start_g(p):
        pltpu.async_copy(wt_hbm.at[idx_v[p]], rows_v[p], gsem[p])

    def wait_g(p):
        pltpu.make_async_copy(wt_hbm.at[idx_v[p]], rows_v[p], gsem[p]).wait()

    def start_o(b, p):
        pltpu.async_copy(
            out_v[p], out_hbm.at[pl.ds(base0 + b * RB, RB)], osem[p])

    def wait_o(b, p):
        pltpu.make_async_copy(
            out_v[p], out_hbm.at[pl.ds(base0 + b * RB, RB)], osem[p]).wait()

    def topk(p):
        def chunk_body(c, st):
            colv = lanes + c * L
            new = []
            for r in range(RB):
                rk, rv = st[2 * r], st[2 * r + 1]
                ck = x_v[p][r, pl.ds(c * L, L)]
                sk, sv = plsc.sort_key_val(ck, colv, descending=False)
                m = rk >= sk
                mk = jnp.where(m, rk, sk)
                mv = jnp.where(m, rv, sv)
                rk, rv = plsc.sort_key_val(mk, mv, descending=True)
                new += [rk, rv]
            return tuple(new)

        init = (neg_inf, zeros_i) * RB
        fin = lax.fori_loop(0, NCHUNK, chunk_body, init)
        for r in range(RB):
            plsc.store_compressed(
                idx_v[p].at[pl.ds(r * K, L)], fin[2 * r + 1], mask=store_mask)

    def accumulate(p):
        def acc_body(j, a):
            for r in range(RB):
                s = rows_v[p][r * K, pl.ds(j * L, L)]
                for k in range(1, K):
                    s = s + rows_v[p][r * K + k, pl.ds(j * L, L)]
                out_v[p][r, pl.ds(j * L, L)] = s
            return a

        lax.fori_loop(0, D // L, acc_body, 0)

    def phase_a(b, p, prefetch):
        wait_x(b, p)
        topk(p)
        start_g(p)
        if prefetch:
            start_x(b + 2, p)

    def phase_b(b, p, wait_out):
        if wait_out:
            wait_o(b - 2, p)
        wait_g(p)
        accumulate(p)
        start_o(b, p)

    start_x(0, 0)
    start_x(1, 1)
    phase_a(0, 0, True)
    phase_a(1, 1, True)
    phase_b(0, 0, False)
    phase_a(2, 0, True)
    phase_b(1, 1, False)

    def main_body(u, carry):
        b1 = 3 + 2 * u
        phase_a(b1, 1, True)
        phase_b(b1 - 1, 0, True)
        phase_a(b1 + 1, 0, True)
        phase_b(b1, 1, True)
        return carry

    lax.fori_loop(0, 29, main_body, 0)

    phase_a(61, 1, True)
    phase_b(60, 0, True)
    phase_a(62, 0, False)
    phase_b(61, 1, True)
    phase_a(63, 1, False)
    phase_b(62, 0, True)
    phase_b(63, 1, True)
    wait_o(62, 0)
    wait_o(63, 1)


def kernel(x, W):
    return _nkq_sc(x, W.T)
